# baseline (speedup 1.0000x reference)
import functools

import jax
import jax.numpy as jnp
from jax import lax
from jax.experimental import pallas as pl
from jax.experimental.pallas import tpu as pltpu
from jax.experimental.pallas import tpu_sc as plsc

_VOCAB = 1000000
_D = 64
_MAX_WAVELENGTH = 10000.0
_B = 1024
_L = 200
_TOTAL = _B * _L

_NC = 2
_NS = 16
_NW = _NC * _NS
_PER_W = _TOTAL // _NW
_SUB = 100
_CHUNK = 200
_NSUB = _CHUNK // _SUB
_NCHUNK = _PER_W // _CHUNK
_IDX_ROWS_PER_W = _PER_W // _SUB


def _enc_body(out_ref):
    pos = lax.broadcasted_iota(jnp.int32, (_L, _D), 0).astype(jnp.float32)
    i = lax.broadcasted_iota(jnp.int32, (_L, _D), 1)
    expo = (2 * (i // 2)).astype(jnp.float32) * (1.0 / _D)
    timescales = jnp.exp(expo * jnp.log(jnp.float32(1.0 / _MAX_WAVELENGTH)))
    angles = pos * timescales
    odd = (i % 2).astype(jnp.float32)
    out_ref[...] = jnp.sin(angles) * (1.0 - odd) + jnp.cos(angles) * odd


def _make_enc():
    return pl.pallas_call(
        _enc_body,
        out_shape=jax.ShapeDtypeStruct((_L, _D), jnp.float32),
    )()


_NBUF = 8


def _sc_body(idx_hbm, table_hbm, enc_hbm, out_hbm, idx_v, enc_v, rows_v, sem):
    wid = lax.axis_index("s") * _NC + lax.axis_index("c")
    idx_row0 = wid * _IDX_ROWS_PER_W
    out_base = wid * _PER_W

    pltpu.sync_copy(idx_hbm.at[pl.ds(idx_row0, _IDX_ROWS_PER_W)], idx_v)
    pltpu.sync_copy(enc_hbm, enc_v)

    for g in range(_NBUF):
        pltpu.async_copy(table_hbm.at[idx_v.at[g]],
                         rows_v.at[pl.ds(g * _SUB, _SUB)], sem)

    @pl.loop(0, _IDX_ROWS_PER_W)
    def _step(g):
        b = lax.rem(g, _NBUF)
        row0 = b * _SUB
        pltpu.make_async_copy(table_hbm.at[idx_v.at[g]],
                              rows_v.at[pl.ds(row0, _SUB)], sem).wait()

        ph = lax.rem(g, _L // _SUB) * _SUB

        @pl.loop(0, _SUB, unroll=4)
        def _add(r):
            for d in range(_D // 16):
                plsc.addupdate(rows_v.at[row0 + r, pl.ds(d * 16, 16)],
                               enc_v[ph + r, pl.ds(d * 16, 16)])

        pltpu.sync_copy(rows_v.at[pl.ds(row0, _SUB)],
                        out_hbm.at[pl.ds(out_base + g * _SUB, _SUB)])

        @pl.when(g < _IDX_ROWS_PER_W - _NBUF)
        def _refill():
            pltpu.async_copy(table_hbm.at[idx_v.at[g + _NBUF]],
                             rows_v.at[pl.ds(row0, _SUB)], sem)


_sc_gather = functools.partial(
    pl.kernel,
    out_type=jax.ShapeDtypeStruct((_TOTAL, _D), jnp.float32),
    mesh=plsc.VectorSubcoreMesh(core_axis_name="c", subcore_axis_name="s"),
    scratch_types=[
        pltpu.VMEM((_IDX_ROWS_PER_W, _SUB), jnp.int32),
        pltpu.VMEM((_L, _D), jnp.float32),
        pltpu.VMEM((_NBUF * _SUB, _D), jnp.float32),
        pltpu.SemaphoreType.DMA,
    ],
    compiler_params=pltpu.CompilerParams(use_tc_tiling_on_sc=False),
)(_sc_body)


def kernel(inputs, table):
    idx2d = inputs.reshape(_TOTAL // _SUB, _SUB).astype(jnp.int32)
    enc = _make_enc()
    out = _sc_gather(idx2d, table, enc)
    return out.reshape(_B, _L, _D)


# speedup vs baseline: 1.0753x; 1.0753x over previous
import functools

import jax
import jax.numpy as jnp
from jax import lax
from jax.experimental import pallas as pl
from jax.experimental.pallas import tpu as pltpu
from jax.experimental.pallas import tpu_sc as plsc

_VOCAB = 1000000
_D = 64
_DP = 128
_MAX_WAVELENGTH = 10000.0
_B = 1024
_L = 200
_TOTAL = _B * _L

_NC = 2
_NS = 16
_NW = _NC * _NS
_PER_W = _TOTAL // _NW
_SUB = 100
_IDX_ROWS_PER_W = _PER_W // _SUB
_NBUF = 8


def _enc_body(out_ref):
    pos = lax.broadcasted_iota(jnp.int32, (_L, _D), 0).astype(jnp.float32)
    i = lax.broadcasted_iota(jnp.int32, (_L, _D), 1)
    expo = (2 * (i // 2)).astype(jnp.float32) * (1.0 / _D)
    timescales = jnp.exp(expo * jnp.log(jnp.float32(1.0 / _MAX_WAVELENGTH)))
    angles = pos * timescales
    odd = (i % 2).astype(jnp.float32)
    out_ref[...] = jnp.sin(angles) * (1.0 - odd) + jnp.cos(angles) * odd


def _make_enc():
    return pl.pallas_call(
        _enc_body,
        out_shape=jax.ShapeDtypeStruct((_L, _D), jnp.float32),
    )()


def _sc_body(idx_hbm, table_hbm, enc_hbm, out_hbm, idx_v, enc_v, rows_v, sem):
    wid = lax.axis_index("s") * _NC + lax.axis_index("c")
    idx_row0 = wid * _IDX_ROWS_PER_W
    out_base = wid * _PER_W

    pltpu.sync_copy(idx_hbm.at[pl.ds(idx_row0, _IDX_ROWS_PER_W)], idx_v)
    pltpu.sync_copy(enc_hbm, enc_v)

    for g in range(_NBUF):
        pltpu.async_copy(table_hbm.at[idx_v.at[g]],
                         rows_v.at[pl.ds(g * _SUB, _SUB)], sem)

    @pl.loop(0, _IDX_ROWS_PER_W)
    def _step(g):
        b = lax.rem(g, _NBUF)
        row0 = b * _SUB
        pltpu.make_async_copy(table_hbm.at[idx_v.at[g]],
                              rows_v.at[pl.ds(row0, _SUB)], sem).wait()

        ph = lax.rem(g, _L // _SUB) * _SUB

        @pl.loop(0, _SUB, unroll=4)
        def _add(r):
            for d in range(_D // 16):
                plsc.addupdate(rows_v.at[row0 + r, pl.ds(d * 16, 16)],
                               enc_v[ph + r, pl.ds(d * 16, 16)])

        pltpu.sync_copy(rows_v.at[pl.ds(row0, _SUB), pl.ds(0, _D)],
                        out_hbm.at[pl.ds(out_base + g * _SUB, _SUB)])

        @pl.when(g < _IDX_ROWS_PER_W - _NBUF)
        def _refill():
            pltpu.async_copy(table_hbm.at[idx_v.at[g + _NBUF]],
                             rows_v.at[pl.ds(row0, _SUB)], sem)


_sc_gather = functools.partial(
    pl.kernel,
    out_type=jax.ShapeDtypeStruct((_TOTAL, _D), jnp.float32),
    mesh=plsc.VectorSubcoreMesh(core_axis_name="c", subcore_axis_name="s"),
    scratch_types=[
        pltpu.VMEM((_IDX_ROWS_PER_W, _SUB), jnp.int32),
        pltpu.VMEM((_L, _D), jnp.float32),
        pltpu.VMEM((_NBUF * _SUB, _DP), jnp.float32),
        pltpu.SemaphoreType.DMA,
    ],
    compiler_params=pltpu.CompilerParams(use_tc_tiling_on_sc=False),
)(_sc_body)


def kernel(inputs, table):
    idx2d = inputs.reshape(_TOTAL // _SUB, _SUB).astype(jnp.int32)
    table_p = jnp.pad(table, ((0, 0), (0, _DP - _D)))
    enc = _make_enc()
    out = _sc_gather(idx2d, table_p, enc)
    return out.reshape(_B, _L, _D)
